# Initial kernel scaffold; baseline (speedup 1.0000x reference)
import jax
import jax.numpy as jnp
from jax.experimental import pallas as pl


def kernel(x, pe):
    raise NotImplementedError("write your pallas kernel here")


# speedup vs baseline: 1.4879x; 1.4879x over previous
import jax
import jax.numpy as jnp
from jax.experimental import pallas as pl


def _add_kernel(x_ref, pe_ref, o_ref):
    o_ref[...] = x_ref[...] + pe_ref[...]


def kernel(x, pe):
    B, S, D = x.shape
    S_BLK = 512
    ns = S // S_BLK
    grid = (ns, B)
    return pl.pallas_call(
        _add_kernel,
        grid=grid,
        in_specs=[
            pl.BlockSpec((1, S_BLK, D), lambda s, b: (b, s, 0)),
            pl.BlockSpec((S_BLK, D), lambda s, b: (s, 0)),
        ],
        out_specs=pl.BlockSpec((1, S_BLK, D), lambda s, b: (b, s, 0)),
        out_shape=jax.ShapeDtypeStruct((B, S, D), x.dtype),
    )(x, pe)


# speedup vs baseline: 1.7335x; 1.1651x over previous
import jax
import jax.numpy as jnp
from jax.experimental import pallas as pl


def _add_kernel(x_ref, pe_ref, o_ref):
    o_ref[...] = x_ref[...] + pe_ref[...]


def kernel(x, pe):
    B, S, D = x.shape
    S_BLK = 2048
    ns = S // S_BLK
    grid = (ns, B)
    return pl.pallas_call(
        _add_kernel,
        grid=grid,
        in_specs=[
            pl.BlockSpec((1, S_BLK, D), lambda s, b: (b, s, 0)),
            pl.BlockSpec((S_BLK, D), lambda s, b: (s, 0)),
        ],
        out_specs=pl.BlockSpec((1, S_BLK, D), lambda s, b: (b, s, 0)),
        out_shape=jax.ShapeDtypeStruct((B, S, D), x.dtype),
    )(x, pe)
